# Initial kernel scaffold; baseline (speedup 1.0000x reference)
import jax
import jax.numpy as jnp
from jax.experimental import pallas as pl


def kernel(legal_mask, Zpos, current_player, current_hash, hash_history, move_count):
    raise NotImplementedError("write your pallas kernel here")


# speedup vs baseline: 67.7908x; 67.7908x over previous
import functools

import jax
import jax.numpy as jnp
from jax import lax
from jax.experimental import pallas as pl
from jax.experimental.pallas import tpu as pltpu
from jax.experimental.pallas import tpu_sc as plsc

B = 16384
N2 = 361
PADN = 368
NGRP = PADN // 16
TBL = 512
NW = 32
GPT = B // NW
CH = 32
NCHUNK = GPT // CH
I32MAX = 2147483647


def _sc_body(hist_hbm, legal_hbm, sd_hbm, rank_hbm, cur_hbm, pl_hbm, mc_hbm,
             out_hbm, sd_v, rank_v, cur_v, pl_v, mc_v, present_v,
             hist_buf, legal_buf, out_buf):
    wid = lax.axis_index("s") * 2 + lax.axis_index("c")
    base = wid * GPT

    pltpu.sync_copy(sd_hbm, sd_v)
    pltpu.sync_copy(rank_hbm, rank_v)
    pltpu.sync_copy(cur_hbm.at[pl.ds(base, GPT)], cur_v.at[pl.ds(0, GPT)])
    pltpu.sync_copy(pl_hbm.at[pl.ds(base, GPT)], pl_v.at[pl.ds(0, GPT)])
    pltpu.sync_copy(mc_hbm.at[pl.ds(base, GPT)], mc_v.at[pl.ds(0, GPT)])

    lanes = lax.iota(jnp.int32, 16)
    for k in range(TBL // 16):
        present_v[pl.ds(k * 16, 16)] = jnp.full((16,), -1, jnp.int32)

    def chunk_body(c, _):
        pltpu.sync_copy(hist_hbm.at[pl.ds(base + c * CH, CH)], hist_buf)
        pltpu.sync_copy(legal_hbm.at[pl.ds(base + c * CH, CH)], legal_buf)

        def game_body(gi, _):
            g = c * CH + gi
            cur = cur_v[pl.ds(g, 16)][0]
            player = lax.bitwise_and(pl_v[pl.ds(g, 16)][0], 1)
            L = lax.min(mc_v[pl.ds(g, 16)][0], N2)
            gvec = jnp.full((16,), g, jnp.int32)
            sd_base = jnp.full((16,), player * TBL, jnp.int32)
            rk_base = jnp.full((16,), player * PADN, jnp.int32)
            ngrp = (L + 15) // 16

            def probe_body(jg, _):
                jbase = jg * 16
                valid = (jbase + lanes) < L
                t = hist_buf[gi, pl.ds(jbase, 16)] ^ cur
                pos = jnp.zeros((16,), jnp.int32)
                for w in (256, 128, 64, 32, 16, 8, 4, 2, 1):
                    v = plsc.load_gather(sd_v, [sd_base + pos + (w - 1)])
                    pos = pos + jnp.where(v <= t, w, 0).astype(jnp.int32)
                hit = jnp.maximum(pos - 1, 0)
                v2 = plsc.load_gather(sd_v, [sd_base + hit])
                found = (pos > 0) & (v2 == t) & valid
                plsc.store_scatter(present_v, [hit], gvec, mask=found)
                return 0

            lax.fori_loop(0, ngrp, probe_body, 0)

            for pg in range(NGRP):
                pts = jnp.full((16,), pg * 16, jnp.int32) + lanes
                r = plsc.load_gather(rank_v, [rk_base + pts])
                rep = plsc.load_gather(present_v, [r]) == gvec
                lg = legal_buf[gi, pl.ds(pg * 16, 16)]
                out_buf[gi, pl.ds(pg * 16, 16)] = jnp.where(
                    (lg > 0) & rep, jnp.float32(0), lg)
            return 0

        lax.fori_loop(0, CH, game_body, 0)
        pltpu.sync_copy(out_buf, out_hbm.at[pl.ds(base + c * CH, CH)])
        return 0

    lax.fori_loop(0, NCHUNK, chunk_body, 0)


def kernel(legal_mask, Zpos, current_player, current_hash, hash_history,
           move_count):
    b, h, w = legal_mask.shape
    d = Zpos[:, 0][None, :] ^ jnp.stack([Zpos[:, 1], Zpos[:, 2]])
    sd = jnp.sort(d, axis=1)
    sd_pad = jnp.concatenate(
        [sd, jnp.full((2, TBL - N2), I32MAX, jnp.int32)], axis=1)
    rank = (jax.vmap(lambda a, v: jnp.searchsorted(a, v, side="right"))(
        sd_pad, d).astype(jnp.int32) - 1)
    rank_pad = jnp.concatenate(
        [rank, jnp.zeros((2, PADN - N2), jnp.int32)], axis=1)

    hist_pad = jnp.pad(hash_history, ((0, 0), (0, PADN - N2)))
    legal_pad = jnp.pad(legal_mask.reshape(b, N2), ((0, 0), (0, PADN - N2)))

    mesh = plsc.VectorSubcoreMesh(core_axis_name="c", subcore_axis_name="s")
    kfn = pl.kernel(
        _sc_body,
        mesh=mesh,
        out_type=jax.ShapeDtypeStruct((B, PADN), jnp.float32),
        compiler_params=pltpu.CompilerParams(needs_layout_passes=False),
        scratch_types=[
            pltpu.VMEM((2 * TBL,), jnp.int32),
            pltpu.VMEM((2 * PADN,), jnp.int32),
            pltpu.VMEM((GPT + 16,), jnp.int32),
            pltpu.VMEM((GPT + 16,), jnp.int32),
            pltpu.VMEM((GPT + 16,), jnp.int32),
            pltpu.VMEM((TBL,), jnp.int32),
            pltpu.VMEM((CH, PADN), jnp.int32),
            pltpu.VMEM((CH, PADN), jnp.float32),
            pltpu.VMEM((CH, PADN), jnp.float32),
        ],
    )
    out = kfn(hist_pad, legal_pad, sd_pad.reshape(-1), rank_pad.reshape(-1),
              current_hash, current_player, move_count)
    return out[:, :N2].reshape(b, h, w)
